# baseline (speedup 1.0000x reference)
import jax
import jax.numpy as jnp
from jax import lax
from jax.experimental import pallas as pl
from jax.experimental.pallas import tpu as pltpu
from jax.experimental.pallas import tpu_sc as plsc

_N = 10000
_D = 128
_NC = 2
_NS = 16
_NW = _NC * _NS
_L = 16
_C = 128
_NCHD = 80
_NCHM = 160
_NCH0 = 116
_EPAD = _NS * _NCHM * _C
_NPAD = 10240
_RPT = _NPAD // _NS
_NBUF = 2

_mesh = plsc.VectorSubcoreMesh(
    core_axis_name="c", subcore_axis_name="s", num_cores=_NC, num_subcores=_NS
)


def _deg_body(slab_hbm, ews_hbm, out_hbm, slab_v, ews_v, zero_v, deg_sh, sem):
    c = lax.axis_index("c")
    s = lax.axis_index("s")
    wid = s * _NC + c

    pltpu.sync_copy(slab_hbm.at[wid], slab_v)
    pltpu.sync_copy(ews_hbm.at[wid], ews_v)

    def zlane(k, _):
        zero_v[pl.ds(k * _L, _L)] = jnp.zeros((_L,), jnp.float32)
        return 0

    lax.fori_loop(0, _C // _L, zlane, 0)
    for k in range(_RPT // _C):
        pltpu.sync_copy(zero_v, deg_sh.at[pl.ds(s * _RPT + k * _C, _C)])
    plsc.subcore_barrier()

    def chunk(i, _):
        pltpu.async_copy(ews_v.at[i], deg_sh.at[slab_v.at[i, 1]], sem, add=True)

        @pl.when(i >= 8)
        def _():
            pltpu.make_async_copy(
                ews_v.at[i - 8], deg_sh.at[slab_v.at[i - 8, 1]], sem
            ).wait()

        return 0

    lax.fori_loop(0, _NCHD, chunk, 0)
    for i in range(_NCHD - 8, _NCHD):
        pltpu.make_async_copy(
            ews_v.at[i], deg_sh.at[slab_v.at[i, 1]], sem
        ).wait()
    plsc.subcore_barrier()
    for k in range(_RPT // _C):
        pltpu.sync_copy(
            deg_sh.at[pl.ds(s * _RPT + k * _C, _C)],
            out_hbm.at[c, pl.ds(s * _RPT + k * _C, _C)],
        )


_sc_deg = pl.kernel(
    _deg_body,
    out_type=jax.ShapeDtypeStruct((_NC, _NPAD), jnp.float32),
    mesh=_mesh,
    scratch_types=[
        pltpu.VMEM((_NCHD, 3, _C), jnp.int32),
        pltpu.VMEM((_NCHD, _C), jnp.float32),
        pltpu.VMEM((_C,), jnp.float32),
        pltpu.VMEM_SHARED((_NPAD,), jnp.float32),
        pltpu.SemaphoreType.DMA,
    ],
)


def _msg_body(y_hbm, slab_hbm, out_hbm, idx3_v, rows_v, acc_sh,
              sg0, sg1, si0, si1, si2, si3):
    c = lax.axis_index("c")
    s = lax.axis_index("s")
    sgs = (sg0, sg1)
    sis = (si0, si1, si2, si3)
    base = c * _NCH0
    count = _NCH0 + c * (_NCHM - 2 * _NCH0)

    def zrow(r, _):
        for k in range(_D // _L):
            rows_v[0, r, pl.ds(k * _L, _L)] = jnp.zeros((_L,), jnp.float32)
        return 0

    lax.fori_loop(0, _C, zrow, 0)
    for k in range(_RPT // _C):
        pltpu.sync_copy(rows_v.at[0], acc_sh.at[pl.ds(s * _RPT + k * _C, _C)])
    plsc.subcore_barrier()

    def idx_start(i, ib):
        pltpu.async_copy(slab_hbm.at[s, base + i], idx3_v.at[ib], sis[ib])

    def idx_wait(i, ib):
        pltpu.make_async_copy(
            slab_hbm.at[s, base + i], idx3_v.at[ib], sis[ib]
        ).wait()

    def gather_start(i, rb, ib):
        pltpu.async_copy(y_hbm.at[idx3_v.at[ib, 0]], rows_v.at[rb], sgs[rb])

    def gather_wait(i, rb, ib):
        pltpu.make_async_copy(
            y_hbm.at[idx3_v.at[ib, 0]], rows_v.at[rb], sgs[rb]
        ).wait()

    def scatter_sync(i, rb, ib):
        pltpu.sync_copy(rows_v.at[rb], acc_sh.at[idx3_v.at[ib, 1]], add=True)

    def compute(i, rb, ib):
        def grp(g, _):
            nv = lax.bitcast_convert_type(
                idx3_v[ib, 2, pl.ds(g * _L, _L)], jnp.float32
            )
            for jj in range(_L):
                bv = lax.broadcast(nv[jj], (_L,))
                e = g * _L + jj
                for k in range(_D // _L):
                    rows_v[rb, e, pl.ds(k * _L, _L)] = (
                        rows_v[rb, e, pl.ds(k * _L, _L)] * bv
                    )
            return 0

        lax.fori_loop(0, _C // _L, grp, 0)

    for j in range(4):
        idx_start(j, j)
    idx_wait(0, 0)
    gather_start(0, 0, 0)
    idx_wait(1, 1)
    gather_start(1, 1, 1)

    def visit(i, rb, ib):
        gather_wait(i, rb, ib)
        compute(i, rb, ib)
        scatter_sync(i, rb, ib)

        @pl.when(i + 2 < count)
        def _():
            idx_wait(i + 2, (ib + 2) % 4)
            gather_start(i + 2, rb, (ib + 2) % 4)

        @pl.when(i + 4 < count)
        def _():
            idx_start(i + 4, ib)

    def group(g, _):
        for j in range(4):
            visit(g * 4 + j, j % _NBUF, j)
        return 0

    lax.fori_loop(0, count // 4, group, 0)
    plsc.subcore_barrier()
    for k in range(_RPT // _C):
        pltpu.sync_copy(
            acc_sh.at[pl.ds(s * _RPT + k * _C, _C)],
            out_hbm.at[c, pl.ds(s * _RPT + k * _C, _C)],
        )


_sc_msg = pl.kernel(
    _msg_body,
    out_type=jax.ShapeDtypeStruct((_NC, _NPAD, _D), jnp.float32),
    mesh=_mesh,
    scratch_types=[
        pltpu.VMEM((4, 3, _C), jnp.int32),
        pltpu.VMEM((_NBUF, _C, _D), jnp.float32),
        pltpu.VMEM_SHARED((_NPAD, _D), jnp.float32),
    ] + [pltpu.SemaphoreType.DMA] * 6,
)

_B = 2048
_G = _NPAD // _B


def _dinv_of(degp_ref):
    off = pl.program_id(0) * _B
    deg = degp_ref[0, pl.ds(off, _B)] + degp_ref[1, pl.ds(off, _B)] + 1.0
    return lax.rsqrt(jnp.maximum(deg, 1e-12))


def _tc_y_body(degp_ref, x_ref, w_ref, y_ref):
    dinv = _dinv_of(degp_ref)
    y_ref[...] = (
        jnp.dot(x_ref[...], w_ref[...], preferred_element_type=jnp.float32)
        * dinv[:, None]
    )


def _tc_mid_body(acc_ref, y_ref, degp_ref, x_ref, b_ref, w_ref, out_ref):
    dinv = _dinv_of(degp_ref)
    h = (acc_ref[0] + acc_ref[1] + y_ref[...]) * dinv[:, None] + b_ref[...] + x_ref[...]
    out_ref[...] = (
        jnp.dot(h, w_ref[...], preferred_element_type=jnp.float32) * dinv[:, None]
    )


def _tc_fin_body(acc_ref, y_ref, degp_ref, x_ref, b_ref, out_ref):
    dinv = _dinv_of(degp_ref)
    h = (acc_ref[0] + acc_ref[1] + y_ref[...]) * dinv[:, None] + b_ref[...]
    out_ref[...] = jnp.maximum(h, 0.0) + x_ref[...]


_spec_deg = pl.BlockSpec((_NC, _NPAD), lambda i: (0, 0))
_spec_row = pl.BlockSpec((_B, _D), lambda i: (i, 0))
_spec_acc = pl.BlockSpec((_NC, _B, _D), lambda i: (0, i, 0))
_spec_w = pl.BlockSpec((_D, _D), lambda i: (0, 0))
_spec_b = pl.BlockSpec((1, _D), lambda i: (0, 0))
_out_row = jax.ShapeDtypeStruct((_NPAD, _D), jnp.float32)


def _tc_y(degp, x, w):
    return pl.pallas_call(
        _tc_y_body,
        grid=(_G,),
        in_specs=[_spec_deg, _spec_row, _spec_w],
        out_specs=_spec_row,
        out_shape=_out_row,
    )(degp, x, w)


def _tc_mid(acc, y, degp, x, b, w):
    return pl.pallas_call(
        _tc_mid_body,
        grid=(_G,),
        in_specs=[_spec_acc, _spec_row, _spec_deg, _spec_row, _spec_b, _spec_w],
        out_specs=_spec_row,
        out_shape=_out_row,
    )(acc, y, degp, x, b, w)


def _tc_fin(acc, y, degp, x, b):
    return pl.pallas_call(
        _tc_fin_body,
        grid=(_G,),
        in_specs=[_spec_acc, _spec_row, _spec_deg, _spec_row, _spec_b],
        out_specs=_spec_row,
        out_shape=_out_row,
    )(acc, y, degp, x, b)


def kernel(latent_variables, edge_index, edge_weight, W1, b1, W2, b2):
    x = jnp.concatenate(
        [latent_variables, jnp.zeros((_NPAD - _N, _D), jnp.float32)]
    )
    src = edge_index[0]
    dst = edge_index[1]
    pad = _EPAD - src.shape[0]
    src_p = jnp.concatenate([src, jnp.zeros((pad,), jnp.int32)])
    dst_p = jnp.concatenate([dst, jnp.zeros((pad,), jnp.int32)])
    ew_p = jnp.concatenate([edge_weight, jnp.zeros((pad,), jnp.float32)])
    ew_i = lax.bitcast_convert_type(ew_p, jnp.int32)
    slab_d = jnp.stack(
        [
            src_p.reshape(_NW, _NCHD, _C),
            dst_p.reshape(_NW, _NCHD, _C),
            ew_i.reshape(_NW, _NCHD, _C),
        ],
        axis=2,
    )
    slab_m = jnp.stack(
        [
            src_p.reshape(_NS, _NCHM, _C),
            dst_p.reshape(_NS, _NCHM, _C),
            ew_i.reshape(_NS, _NCHM, _C),
        ],
        axis=2,
    )
    ews = ew_p.reshape(_NW, _NCHD, _C)

    degp = _sc_deg(slab_d, ews)
    y1 = _tc_y(degp, x, W1)
    acc1 = _sc_msg(y1, slab_m)
    y2 = _tc_mid(acc1, y1, degp, x, b1.reshape(1, _D), W2)
    acc2 = _sc_msg(y2, slab_m)
    out = _tc_fin(acc2, y2, degp, x, b2.reshape(1, _D))
    return out[:_N]


# speedup vs baseline: 1.0524x; 1.0524x over previous
import jax
import jax.numpy as jnp
from jax import lax
from jax.experimental import pallas as pl
from jax.experimental.pallas import tpu as pltpu
from jax.experimental.pallas import tpu_sc as plsc

_N = 10000
_D = 128
_NC = 2
_NS = 16
_NW = _NC * _NS
_L = 16
_C = 128
_NCHD = 80
_NCHM = 160
_NCH0 = 120
_EPAD = _NS * _NCHM * _C
_NPAD = 10240
_RPT = _NPAD // _NS
_NBUF = 2

_mesh = plsc.VectorSubcoreMesh(
    core_axis_name="c", subcore_axis_name="s", num_cores=_NC, num_subcores=_NS
)


def _deg_body(slab_hbm, ews_hbm, out_hbm, slab_v, ews_v, zero_v, deg_sh, sem):
    c = lax.axis_index("c")
    s = lax.axis_index("s")
    wid = s * _NC + c

    pltpu.sync_copy(slab_hbm.at[wid], slab_v)
    pltpu.sync_copy(ews_hbm.at[wid], ews_v)

    def zlane(k, _):
        zero_v[pl.ds(k * _L, _L)] = jnp.zeros((_L,), jnp.float32)
        return 0

    lax.fori_loop(0, _C // _L, zlane, 0)
    for k in range(_RPT // _C):
        pltpu.sync_copy(zero_v, deg_sh.at[pl.ds(s * _RPT + k * _C, _C)])
    plsc.subcore_barrier()

    def chunk(i, _):
        pltpu.async_copy(ews_v.at[i], deg_sh.at[slab_v.at[i, 1]], sem, add=True)

        @pl.when(i >= 8)
        def _():
            pltpu.make_async_copy(
                ews_v.at[i - 8], deg_sh.at[slab_v.at[i - 8, 1]], sem
            ).wait()

        return 0

    lax.fori_loop(0, _NCHD, chunk, 0)
    for i in range(_NCHD - 8, _NCHD):
        pltpu.make_async_copy(
            ews_v.at[i], deg_sh.at[slab_v.at[i, 1]], sem
        ).wait()
    plsc.subcore_barrier()
    for k in range(_RPT // _C):
        pltpu.sync_copy(
            deg_sh.at[pl.ds(s * _RPT + k * _C, _C)],
            out_hbm.at[c, pl.ds(s * _RPT + k * _C, _C)],
        )


_sc_deg = pl.kernel(
    _deg_body,
    out_type=jax.ShapeDtypeStruct((_NC, _NPAD), jnp.float32),
    mesh=_mesh,
    scratch_types=[
        pltpu.VMEM((_NCHD, 3, _C), jnp.int32),
        pltpu.VMEM((_NCHD, _C), jnp.float32),
        pltpu.VMEM((_C,), jnp.float32),
        pltpu.VMEM_SHARED((_NPAD,), jnp.float32),
        pltpu.SemaphoreType.DMA,
    ],
)


def _msg_body(y_hbm, slab_hbm, out_hbm, idx3_v, rows_v, acc_sh,
              sg0, sg1, si0, si1, si2, si3):
    c = lax.axis_index("c")
    s = lax.axis_index("s")
    sgs = (sg0, sg1)
    sis = (si0, si1, si2, si3)
    base = c * _NCH0
    count = _NCH0 + c * (_NCHM - 2 * _NCH0)

    def zrow(r, _):
        for k in range(_D // _L):
            rows_v[0, r, pl.ds(k * _L, _L)] = jnp.zeros((_L,), jnp.float32)
        return 0

    lax.fori_loop(0, _C, zrow, 0)
    for k in range(_RPT // _C):
        pltpu.sync_copy(rows_v.at[0], acc_sh.at[pl.ds(s * _RPT + k * _C, _C)])
    plsc.subcore_barrier()

    def idx_start(i, ib):
        pltpu.async_copy(slab_hbm.at[s, base + i], idx3_v.at[ib], sis[ib])

    def idx_wait(i, ib):
        pltpu.make_async_copy(
            slab_hbm.at[s, base + i], idx3_v.at[ib], sis[ib]
        ).wait()

    def gather_start(i, rb, ib):
        pltpu.async_copy(y_hbm.at[idx3_v.at[ib, 0]], rows_v.at[rb], sgs[rb])

    def gather_wait(i, rb, ib):
        pltpu.make_async_copy(
            y_hbm.at[idx3_v.at[ib, 0]], rows_v.at[rb], sgs[rb]
        ).wait()

    def scatter_sync(i, rb, ib):
        pltpu.sync_copy(rows_v.at[rb], acc_sh.at[idx3_v.at[ib, 1]], add=True)

    def compute(i, rb, ib):
        def grp(g, _):
            nv = lax.bitcast_convert_type(
                idx3_v[ib, 2, pl.ds(g * _L, _L)], jnp.float32
            )
            for jj in range(_L):
                bv = lax.broadcast(nv[jj], (_L,))
                e = g * _L + jj
                for k in range(_D // _L):
                    rows_v[rb, e, pl.ds(k * _L, _L)] = (
                        rows_v[rb, e, pl.ds(k * _L, _L)] * bv
                    )
            return 0

        lax.fori_loop(0, _C // _L, grp, 0)

    for j in range(4):
        idx_start(j, j)
    idx_wait(0, 0)
    gather_start(0, 0, 0)
    idx_wait(1, 1)
    gather_start(1, 1, 1)

    def visit(i, rb, ib):
        gather_wait(i, rb, ib)
        compute(i, rb, ib)
        scatter_sync(i, rb, ib)

        @pl.when(i + 2 < count)
        def _():
            idx_wait(i + 2, (ib + 2) % 4)
            gather_start(i + 2, rb, (ib + 2) % 4)

        @pl.when(i + 4 < count)
        def _():
            idx_start(i + 4, ib)

    def group(g, _):
        for j in range(4):
            visit(g * 4 + j, j % _NBUF, j)
        return 0

    lax.fori_loop(0, count // 4, group, 0)
    plsc.subcore_barrier()
    for k in range(_RPT // _C):
        pltpu.sync_copy(
            acc_sh.at[pl.ds(s * _RPT + k * _C, _C)],
            out_hbm.at[c, pl.ds(s * _RPT + k * _C, _C)],
        )


_sc_msg = pl.kernel(
    _msg_body,
    out_type=jax.ShapeDtypeStruct((_NC, _NPAD, _D), jnp.float32),
    mesh=_mesh,
    scratch_types=[
        pltpu.VMEM((4, 3, _C), jnp.int32),
        pltpu.VMEM((_NBUF, _C, _D), jnp.float32),
        pltpu.VMEM_SHARED((_NPAD, _D), jnp.float32),
    ] + [pltpu.SemaphoreType.DMA] * 6,
)

_B = 2048
_G = _NPAD // _B


def _dinv_of(degp_ref):
    off = pl.program_id(0) * _B
    deg = degp_ref[0, pl.ds(off, _B)] + degp_ref[1, pl.ds(off, _B)] + 1.0
    return lax.rsqrt(jnp.maximum(deg, 1e-12))


def _tc_y_body(degp_ref, x_ref, w_ref, y_ref):
    dinv = _dinv_of(degp_ref)
    y_ref[...] = (
        jnp.dot(x_ref[...], w_ref[...], preferred_element_type=jnp.float32)
        * dinv[:, None]
    )


def _tc_mid_body(acc_ref, y_ref, degp_ref, x_ref, b_ref, w_ref, out_ref):
    dinv = _dinv_of(degp_ref)
    h = (acc_ref[0] + acc_ref[1] + y_ref[...]) * dinv[:, None] + b_ref[...] + x_ref[...]
    out_ref[...] = (
        jnp.dot(h, w_ref[...], preferred_element_type=jnp.float32) * dinv[:, None]
    )


def _tc_fin_body(acc_ref, y_ref, degp_ref, x_ref, b_ref, out_ref):
    dinv = _dinv_of(degp_ref)
    h = (acc_ref[0] + acc_ref[1] + y_ref[...]) * dinv[:, None] + b_ref[...]
    out_ref[...] = jnp.maximum(h, 0.0) + x_ref[...]


_spec_deg = pl.BlockSpec((_NC, _NPAD), lambda i: (0, 0))
_spec_row = pl.BlockSpec((_B, _D), lambda i: (i, 0))
_spec_acc = pl.BlockSpec((_NC, _B, _D), lambda i: (0, i, 0))
_spec_w = pl.BlockSpec((_D, _D), lambda i: (0, 0))
_spec_b = pl.BlockSpec((1, _D), lambda i: (0, 0))
_out_row = jax.ShapeDtypeStruct((_NPAD, _D), jnp.float32)


def _tc_y(degp, x, w):
    return pl.pallas_call(
        _tc_y_body,
        grid=(_G,),
        in_specs=[_spec_deg, _spec_row, _spec_w],
        out_specs=_spec_row,
        out_shape=_out_row,
    )(degp, x, w)


def _tc_mid(acc, y, degp, x, b, w):
    return pl.pallas_call(
        _tc_mid_body,
        grid=(_G,),
        in_specs=[_spec_acc, _spec_row, _spec_deg, _spec_row, _spec_b, _spec_w],
        out_specs=_spec_row,
        out_shape=_out_row,
    )(acc, y, degp, x, b, w)


def _tc_fin(acc, y, degp, x, b):
    return pl.pallas_call(
        _tc_fin_body,
        grid=(_G,),
        in_specs=[_spec_acc, _spec_row, _spec_deg, _spec_row, _spec_b],
        out_specs=_spec_row,
        out_shape=_out_row,
    )(acc, y, degp, x, b)


def kernel(latent_variables, edge_index, edge_weight, W1, b1, W2, b2):
    x = jnp.concatenate(
        [latent_variables, jnp.zeros((_NPAD - _N, _D), jnp.float32)]
    )
    src = edge_index[0]
    dst = edge_index[1]
    pad = _EPAD - src.shape[0]
    src_p = jnp.concatenate([src, jnp.zeros((pad,), jnp.int32)])
    dst_p = jnp.concatenate([dst, jnp.zeros((pad,), jnp.int32)])
    ew_p = jnp.concatenate([edge_weight, jnp.zeros((pad,), jnp.float32)])
    ew_i = lax.bitcast_convert_type(ew_p, jnp.int32)
    slab_d = jnp.stack(
        [
            src_p.reshape(_NW, _NCHD, _C),
            dst_p.reshape(_NW, _NCHD, _C),
            ew_i.reshape(_NW, _NCHD, _C),
        ],
        axis=2,
    )
    slab_m = jnp.stack(
        [
            src_p.reshape(_NS, _NCHM, _C),
            dst_p.reshape(_NS, _NCHM, _C),
            ew_i.reshape(_NS, _NCHM, _C),
        ],
        axis=2,
    )
    ews = ew_p.reshape(_NW, _NCHD, _C)

    degp = _sc_deg(slab_d, ews)
    y1 = _tc_y(degp, x, W1)
    acc1 = _sc_msg(y1, slab_m)
    y2 = _tc_mid(acc1, y1, degp, x, b1.reshape(1, _D), W2)
    acc2 = _sc_msg(y2, slab_m)
    out = _tc_fin(acc2, y2, degp, x, b2.reshape(1, _D))
    return out[:_N]
